# baseline (speedup 1.0000x reference)
import functools

import jax
import jax.numpy as jnp
from jax import lax
from jax.experimental import pallas as pl
from jax.experimental.pallas import tpu as pltpu
from jax.experimental.pallas import tpu_sc as plsc

STD_ = 3.0
MS = 6
B, T, C = 64, 2048, 256
NC, NS = 2, 16
NW = NC * NS
BPW = B // NW
R = 64
NCHUNK = T // R
NR = 256
H = 8
L = 16
NG = C // L

_mesh = plsc.VectorSubcoreMesh(core_axis_name="core", subcore_axis_name="sub")


@functools.partial(
    pl.kernel,
    out_type=jax.ShapeDtypeStruct((B, T, C), jnp.float32),
    mesh=_mesh,
    scratch_types=[
        pltpu.VMEM((C,), jnp.int32),
        pltpu.VMEM((NR, C), jnp.float32),
        pltpu.VMEM((R, C), jnp.float32),
        pltpu.VMEM((R, C), jnp.float32),
        pltpu.SemaphoreType.DMA((2,)),
        pltpu.SemaphoreType.DMA((2,)),
    ],
    compiler_params=pltpu.CompilerParams(needs_layout_passes=False,
                                         use_tc_tiling_on_sc=True),
)
def _shift_sc(data_hbm, wbase_hbm, out_hbm, wbase_v, ring, out_0, out_1,
              sin, sout):
    outs = [out_0, out_1]
    wid = lax.axis_index("sub") * NC + lax.axis_index("core")

    for bi in range(BPW):
        b = wid * BPW + bi
        pltpu.sync_copy(wbase_hbm.at[b], wbase_v)
        wregs = [wbase_v[pl.ds(g * L, L)] for g in range(NG)]
        cregs = [lax.iota(jnp.int32, L) + g * L for g in range(NG)]

        def load_chunk(ci):
            pltpu.async_copy(data_hbm.at[b, pl.ds(ci * R, R)],
                             ring.at[pl.ds((ci * R) % NR, R)],
                             sin.at[ci % 2])

        pltpu.async_copy(data_hbm.at[b, pl.ds(T - H, H)],
                         ring.at[pl.ds(NR - H, H)], sin.at[1])
        load_chunk(0)
        load_chunk(1)

        def chunk_pair(ci2, carry):
            for k in range(2):
                ci = ci2 * 2 + k
                t0 = ci * R

                @pl.when(ci == 0)
                def _():
                    pltpu.make_async_copy(data_hbm.at[b, pl.ds(0, H)],
                                          ring.at[pl.ds(0, H)],
                                          sin.at[1]).wait()
                    pltpu.make_async_copy(data_hbm.at[b, pl.ds(0, R)],
                                          ring.at[pl.ds(0, R)],
                                          sin.at[0]).wait()

                @pl.when(ci < NCHUNK - 1)
                def _():
                    pltpu.make_async_copy(data_hbm.at[b, pl.ds(0, R)],
                                          ring.at[pl.ds(0, R)],
                                          sin.at[(ci + 1) % 2]).wait()

                @pl.when(ci == NCHUNK - 1)
                def _():
                    pltpu.make_async_copy(data_hbm.at[b, pl.ds(0, H)],
                                          ring.at[pl.ds(0, H)],
                                          sin.at[(ci + 1) % 2]).wait()

                @pl.when(ci + 2 < NCHUNK)
                def _():
                    load_chunk(ci + 2)

                @pl.when(ci + 2 == NCHUNK)
                def _():
                    pltpu.async_copy(data_hbm.at[b, pl.ds(0, H)],
                                     ring.at[pl.ds(0, H)], sin.at[ci % 2])

                @pl.when(ci >= 2)
                def _():
                    pltpu.make_async_copy(outs[k],
                                          out_hbm.at[b, pl.ds(0, R)],
                                          sout.at[k]).wait()

                def row_body(r, c2):
                    t_abs = t0 + r
                    vals = [plsc.load_gather(
                                ring, [(wregs[g] + t_abs) & (NR - 1), cregs[g]])
                            for g in range(NG)]
                    for g in range(NG):
                        outs[k][r, pl.ds(g * L, L)] = vals[g]
                    return c2

                lax.fori_loop(0, R, row_body, 0, unroll=8)
                pltpu.async_copy(outs[k], out_hbm.at[b, pl.ds(t0, R)],
                                 sout.at[k])
            return carry

        lax.fori_loop(0, NCHUNK // 2, chunk_pair, 0)
        for k in range(2):
            pltpu.make_async_copy(outs[k], out_hbm.at[b, pl.ds(0, R)],
                                  sout.at[k]).wait()


def kernel(data):
    skey = jax.random.key(42)
    shifts = jax.random.normal(skey, (B, 1, C), dtype=jnp.float32) * STD_
    shifts = jnp.clip(jnp.round(shifts).astype(jnp.int32), -MS, MS)[:, 0, :]
    wbase = -shifts
    return _shift_sc(data, wbase)


# speedup vs baseline: 1.0155x; 1.0155x over previous
import functools

import jax
import jax.numpy as jnp
from jax import lax
from jax.experimental import pallas as pl
from jax.experimental.pallas import tpu as pltpu
from jax.experimental.pallas import tpu_sc as plsc

STD_ = 3.0
MS = 6
B, T, C = 64, 2048, 256
NC, NS = 2, 16
NW = NC * NS
BPW = B // NW
R = 64
NCHUNK = T // R
NR = 256
H = 8
L = 16
NG = C // L

_mesh = plsc.VectorSubcoreMesh(core_axis_name="core", subcore_axis_name="sub")


@functools.partial(
    pl.kernel,
    out_type=jax.ShapeDtypeStruct((B, T, C), jnp.float32),
    mesh=_mesh,
    scratch_types=[
        pltpu.VMEM((C,), jnp.int32),
        pltpu.VMEM((NR, C), jnp.float32),
        pltpu.VMEM((R, C), jnp.float32),
        pltpu.VMEM((R, C), jnp.float32),
        pltpu.SemaphoreType.DMA((2,)),
        pltpu.SemaphoreType.DMA((2,)),
    ],
    compiler_params=pltpu.CompilerParams(needs_layout_passes=False,
                                         use_tc_tiling_on_sc=True),
)
def _shift_sc(data_hbm, wbase_hbm, out_hbm, wbase_v, ring, out_0, out_1,
              sin, sout):
    outs = [out_0, out_1]
    wid = lax.axis_index("sub") * NC + lax.axis_index("core")

    for bi in range(BPW):
        b = wid * BPW + bi
        pltpu.sync_copy(wbase_hbm.at[b], wbase_v)
        wregs = [wbase_v[pl.ds(g * L, L)] for g in range(NG)]
        cregs = [lax.iota(jnp.int32, L) + g * L for g in range(NG)]

        def load_chunk(ci):
            pltpu.async_copy(data_hbm.at[b, pl.ds(ci * R, R)],
                             ring.at[pl.ds((ci * R) % NR, R)],
                             sin.at[ci % 2])

        pltpu.async_copy(data_hbm.at[b, pl.ds(T - H, H)],
                         ring.at[pl.ds(NR - H, H)], sin.at[1])
        load_chunk(0)
        load_chunk(1)

        def chunk_pair(ci2, carry):
            for k in range(2):
                ci = ci2 * 2 + k
                t0 = ci * R

                @pl.when(ci == 0)
                def _():
                    pltpu.make_async_copy(data_hbm.at[b, pl.ds(0, H)],
                                          ring.at[pl.ds(0, H)],
                                          sin.at[1]).wait()
                    pltpu.make_async_copy(data_hbm.at[b, pl.ds(0, R)],
                                          ring.at[pl.ds(0, R)],
                                          sin.at[0]).wait()

                @pl.when(ci < NCHUNK - 1)
                def _():
                    pltpu.make_async_copy(data_hbm.at[b, pl.ds(0, R)],
                                          ring.at[pl.ds(0, R)],
                                          sin.at[(ci + 1) % 2]).wait()

                @pl.when(ci == NCHUNK - 1)
                def _():
                    pltpu.make_async_copy(data_hbm.at[b, pl.ds(0, H)],
                                          ring.at[pl.ds(0, H)],
                                          sin.at[(ci + 1) % 2]).wait()

                @pl.when(ci + 2 < NCHUNK)
                def _():
                    load_chunk(ci + 2)

                @pl.when(ci + 2 == NCHUNK)
                def _():
                    pltpu.async_copy(data_hbm.at[b, pl.ds(0, H)],
                                     ring.at[pl.ds(0, H)], sin.at[ci % 2])

                @pl.when(ci >= 2)
                def _():
                    pltpu.make_async_copy(outs[k],
                                          out_hbm.at[b, pl.ds(0, R)],
                                          sout.at[k]).wait()

                def row_body(r, c2):
                    t_abs = t0 + r
                    vals = [plsc.load_gather(
                                ring, [(wregs[g] + t_abs) & (NR - 1), cregs[g]])
                            for g in range(NG)]
                    for g in range(NG):
                        outs[k][r, pl.ds(g * L, L)] = vals[g]
                    return c2

                lax.fori_loop(0, R, row_body, 0, unroll=4)
                pltpu.async_copy(outs[k], out_hbm.at[b, pl.ds(t0, R)],
                                 sout.at[k])
            return carry

        lax.fori_loop(0, NCHUNK // 2, chunk_pair, 0)
        for k in range(2):
            pltpu.make_async_copy(outs[k], out_hbm.at[b, pl.ds(0, R)],
                                  sout.at[k]).wait()


def kernel(data):
    skey = jax.random.key(42)
    shifts = jax.random.normal(skey, (B, 1, C), dtype=jnp.float32) * STD_
    shifts = jnp.clip(jnp.round(shifts).astype(jnp.int32), -MS, MS)[:, 0, :]
    wbase = -shifts
    return _shift_sc(data, wbase)
